# Initial kernel scaffold; baseline (speedup 1.0000x reference)
import functools

import jax
import jax.numpy as jnp
from jax import lax
from jax.experimental import pallas as pl
from jax.experimental.pallas import tpu as pltpu
from jax.experimental.pallas import tpu_sc as plsc

N = 10000
E = 320000
D = 128
DP = 144
NSUB = 16
B = 128
NB = 157
E_PAD = NSUB * NB * B
ROWS_PAD = N + 16
N_TILE = N // NSUB
Z_TILE = ROWS_PAD // NSUB


def _sc_segment_sums(x_aug, edges):
    mesh = plsc.VectorSubcoreMesh(core_axis_name="c", subcore_axis_name="s")

    @functools.partial(
        pl.kernel,
        mesh=mesh,
        out_type=jax.ShapeDtypeStruct((2, N, DP), jnp.float32),
        scratch_types=[
            pltpu.VMEM((B,), jnp.int32),
            pltpu.VMEM((B,), jnp.int32),
            pltpu.VMEM((B, DP), jnp.float32),
            pltpu.VMEM_SHARED((ROWS_PAD, DP), jnp.float32),
            pltpu.SemaphoreType.DMA,
        ],
    )
    def k(x_hbm, e_hbm, out_hbm, gidx, sidx, rows, acc, sem):
        c = lax.axis_index("c")
        s = lax.axis_index("s")
        zvec = jnp.zeros((16,), jnp.float32)

        def zrow(i, carry):
            for kk in range(DP // 16):
                rows[i, pl.ds(kk * 16, 16)] = zvec
            return carry
        lax.fori_loop(0, B, zrow, 0)
        zbase = s * Z_TILE
        for j in range(Z_TILE // B):
            pltpu.sync_copy(rows, acc.at[pl.ds(zbase + j * B, B)])
        rem = Z_TILE - (Z_TILE // B) * B
        if rem:
            pltpu.sync_copy(rows.at[pl.ds(0, rem)],
                            acc.at[pl.ds(zbase + (Z_TILE // B) * B, rem)])
        plsc.subcore_barrier()

        ebase = s * NB * B

        def body(i, carry):
            base = ebase + i * B
            pltpu.sync_copy(e_hbm.at[c, pl.ds(base, B)], gidx)
            pltpu.sync_copy(e_hbm.at[1 - c, pl.ds(base, B)], sidx)
            pltpu.async_copy(x_hbm.at[gidx], rows, sem).wait()
            pltpu.sync_copy(rows, acc.at[sidx], add=True)
            return carry
        lax.fori_loop(0, NB, body, 0)
        plsc.subcore_barrier()

        obase = s * N_TILE
        pltpu.sync_copy(acc.at[pl.ds(obase, N_TILE)],
                        out_hbm.at[c, pl.ds(obase, N_TILE)])

    return k(x_aug, edges)


def _tc_combine(sums, x, Wlt, Wrt, Wdt, b_l, b_d):
    BN = 1000
    grid = (N // BN,)

    def body(a_ref, x_ref, wl_ref, wr_ref, wd_ref, bl_ref, bd_ref, o_ref):
        a0 = a_ref[0]
        a1 = a_ref[1]
        xb = x_ref[...]
        afwd = a0[:, :D]
        arev = a1[:, :D]
        din = jnp.sum(a0[:, D:], axis=1, keepdims=True)
        dout = jnp.sum(a1[:, D:], axis=1, keepdims=True)
        agg = afwd / jnp.maximum(din, 1.0)
        hp = jax.lax.Precision.HIGHEST
        h = jnp.maximum(jnp.dot(agg, wl_ref[...], precision=hp)
                        + jnp.dot(xb, wr_ref[...], precision=hp)
                        + bl_ref[...], 0.0)
        o_ref[...] = (h + jnp.dot(arev - dout * xb, wd_ref[...], precision=hp)
                      + dout * bd_ref[...])

    return pl.pallas_call(
        body,
        grid=grid,
        in_specs=[
            pl.BlockSpec((2, BN, DP), lambda i: (0, i, 0)),
            pl.BlockSpec((BN, D), lambda i: (i, 0)),
            pl.BlockSpec((D, D), lambda i: (0, 0)),
            pl.BlockSpec((D, D), lambda i: (0, 0)),
            pl.BlockSpec((D, D), lambda i: (0, 0)),
            pl.BlockSpec((1, D), lambda i: (0, 0)),
            pl.BlockSpec((1, D), lambda i: (0, 0)),
        ],
        out_specs=pl.BlockSpec((BN, D), lambda i: (i, 0)),
        out_shape=jax.ShapeDtypeStruct((N, D), jnp.float32),
    )(sums, x, Wlt, Wrt, Wdt, b_l, b_d)


def kernel(x, edge_index, W_l, b_l, W_r, W_d, b_d):
    x = x.astype(jnp.float32)
    x_aug = jnp.concatenate(
        [x, jnp.ones((N, 1), jnp.float32), jnp.zeros((N, DP - D - 1), jnp.float32)],
        axis=1)
    x_aug = jnp.concatenate([x_aug, jnp.zeros((ROWS_PAD - N, DP), jnp.float32)],
                            axis=0)
    e = edge_index.astype(jnp.int32)
    e_pad = jnp.concatenate(
        [e, jnp.full((2, E_PAD - E), N, jnp.int32)], axis=1)

    sums = _sc_segment_sums(x_aug, e_pad)
    return _tc_combine(sums, x, W_l.T, W_r.T, W_d.T, b_l[None, :], b_d[None, :])


# speedup vs baseline: 5.5506x; 5.5506x over previous
import functools

import jax
import jax.numpy as jnp
from jax import lax
from jax.experimental import pallas as pl
from jax.experimental.pallas import tpu as pltpu
from jax.experimental.pallas import tpu_sc as plsc

N = 10000
E = 320000
D = 128
NSUB = 16
B = 128
NB = 157
E_PAD = NSUB * NB * B
NPAD = 10240
R_TILE = NPAD // NSUB


def _sc_segment_sums(x_pad, edges):
    mesh = plsc.VectorSubcoreMesh(core_axis_name="c", subcore_axis_name="s")

    @functools.partial(
        pl.kernel,
        mesh=mesh,
        out_type=(jax.ShapeDtypeStruct((2, NPAD, D), jnp.float32),
                  jax.ShapeDtypeStruct((2, NPAD), jnp.float32)),
        scratch_types=[
            pltpu.VMEM((B,), jnp.int32),
            pltpu.VMEM((B,), jnp.int32),
            pltpu.VMEM((B, D), jnp.float32),
            pltpu.VMEM((B,), jnp.float32),
            pltpu.VMEM((R_TILE,), jnp.float32),
            pltpu.VMEM_SHARED((NPAD, D), jnp.float32),
            pltpu.VMEM_SHARED((NPAD,), jnp.float32),
            pltpu.SemaphoreType.DMA,
        ],
    )
    def k(x_hbm, e_hbm, out_hbm, deg_hbm,
          gidx, sidx, rows, ones, zed, acc, dacc, sem):
        c = lax.axis_index("c")
        s = lax.axis_index("s")
        zvec = jnp.zeros((16,), jnp.float32)
        ovec = jnp.ones((16,), jnp.float32)

        def zrow(i, carry):
            for kk in range(D // 16):
                rows[i, pl.ds(kk * 16, 16)] = zvec
            return carry
        lax.fori_loop(0, B, zrow, 0)
        for j in range(B // 16):
            ones[pl.ds(j * 16, 16)] = ovec
        def zdeg(i, carry):
            zed[pl.ds(i * 16, 16)] = zvec
            return carry
        lax.fori_loop(0, R_TILE // 16, zdeg, 0)
        rbase = s * R_TILE
        for j in range(R_TILE // B):
            pltpu.sync_copy(rows, acc.at[pl.ds(rbase + j * B, B)])
        pltpu.sync_copy(zed, dacc.at[pl.ds(rbase, R_TILE)])
        plsc.subcore_barrier()

        ebase = s * NB * B

        def body(i, carry):
            base = ebase + i * B
            pltpu.sync_copy(e_hbm.at[c, pl.ds(base, B)], gidx)
            pltpu.sync_copy(e_hbm.at[1 - c, pl.ds(base, B)], sidx)
            pltpu.async_copy(x_hbm.at[gidx], rows, sem).wait()
            pltpu.sync_copy(rows, acc.at[sidx], add=True)
            pltpu.sync_copy(ones, dacc.at[sidx], add=True)
            return carry
        lax.fori_loop(0, NB, body, 0)
        plsc.subcore_barrier()

        pltpu.sync_copy(acc.at[pl.ds(rbase, R_TILE)],
                        out_hbm.at[c, pl.ds(rbase, R_TILE)])
        pltpu.sync_copy(dacc.at[pl.ds(rbase, R_TILE)],
                        deg_hbm.at[c, pl.ds(rbase, R_TILE)])

    return k(x_pad, edges)


def _tc_combine(sums, degs, x_pad, Wlt, Wrt, Wdt, b_l, b_d):
    BN = 1024
    grid = (NPAD // BN,)

    def body(a_ref, d_ref, x_ref, wl_ref, wr_ref, wd_ref, bl_ref, bd_ref,
             o_ref):
        afwd = a_ref[0]
        arev = a_ref[1]
        din = d_ref[0]
        dout = d_ref[1]
        xb = x_ref[...]
        agg = afwd / jnp.maximum(din, 1.0)
        hp = jax.lax.Precision.HIGHEST
        h = jnp.maximum(jnp.dot(agg, wl_ref[...], precision=hp)
                        + jnp.dot(xb, wr_ref[...], precision=hp)
                        + bl_ref[...], 0.0)
        o_ref[...] = (h + jnp.dot(arev - dout * xb, wd_ref[...], precision=hp)
                      + dout * bd_ref[...])

    return pl.pallas_call(
        body,
        grid=grid,
        in_specs=[
            pl.BlockSpec((2, BN, D), lambda i: (0, i, 0)),
            pl.BlockSpec((2, BN, 1), lambda i: (0, i, 0)),
            pl.BlockSpec((BN, D), lambda i: (i, 0)),
            pl.BlockSpec((D, D), lambda i: (0, 0)),
            pl.BlockSpec((D, D), lambda i: (0, 0)),
            pl.BlockSpec((D, D), lambda i: (0, 0)),
            pl.BlockSpec((1, D), lambda i: (0, 0)),
            pl.BlockSpec((1, D), lambda i: (0, 0)),
        ],
        out_specs=pl.BlockSpec((BN, D), lambda i: (i, 0)),
        out_shape=jax.ShapeDtypeStruct((NPAD, D), jnp.float32),
    )(sums, degs, x_pad, Wlt, Wrt, Wdt, b_l, b_d)


def kernel(x, edge_index, W_l, b_l, W_r, W_d, b_d):
    x = x.astype(jnp.float32)
    x_pad = jnp.concatenate(
        [x, jnp.zeros((NPAD - N, D), jnp.float32)], axis=0)
    e = edge_index.astype(jnp.int32)
    e_pad = jnp.concatenate(
        [e, jnp.full((2, E_PAD - E), N, jnp.int32)], axis=1)

    sums, degs = _sc_segment_sums(x_pad, e_pad)
    out = _tc_combine(sums, degs[:, :, None], x_pad,
                      W_l.T, W_r.T, W_d.T, b_l[None, :], b_d[None, :])
    return out[:N]
